# Initial kernel scaffold; baseline (speedup 1.0000x reference)
import jax
import jax.numpy as jnp
from jax.experimental import pallas as pl


def kernel(x, edge_index, W1, b1, W2, b2, l0f_Wih, l0f_Whh, l0f_bih, l0f_bhh, l0b_Wih, l0b_Whh, l0b_bih, l0b_bhh, l1f_Wih, l1f_Whh, l1f_bih, l1f_bhh, l1b_Wih, l1b_Whh, l1b_bih, l1b_bhh, Wl, bl):
    raise NotImplementedError("write your pallas kernel here")


# speedup vs baseline: 5.1402x; 5.1402x over previous
import functools

import jax
import jax.numpy as jnp
from jax import lax
from jax.experimental import pallas as pl
from jax.experimental.pallas import tpu as pltpu
from jax.experimental.pallas import tpu_sc as plsc

N = 10000
E = 320000
NWORK = 32
CHUNK = 128
EPW = (E + NWORK * CHUNK - 1) // (NWORK * CHUNK) * CHUNK
E_PAD = EPW * NWORK
NCHUNK = EPW // CHUNK
NPAD = 10240
RPT = NPAD // 16

def _mesh():
    return plsc.VectorSubcoreMesh(core_axis_name="c", subcore_axis_name="s")


_HIGH = jax.lax.Precision.HIGHEST


def _dot(a, b):
    return jnp.dot(a, b, precision=_HIGH, preferred_element_type=jnp.float32)


def _sc_deg_body(dst_hbm, ones_hbm, zer_hbm, out_hbm, dst_v, ones_v, acc):
    cid = lax.axis_index("c")
    sid = lax.axis_index("s")
    wid = cid * 16 + sid
    pltpu.sync_copy(dst_hbm.at[wid], dst_v)
    pltpu.sync_copy(ones_hbm, ones_v)
    pltpu.sync_copy(zer_hbm, acc.at[pl.ds(sid * RPT, RPT)])
    plsc.subcore_barrier()

    def body(c, carry):
        pltpu.sync_copy(ones_v, acc.at[dst_v.at[c]], add=True)
        return carry

    lax.fori_loop(0, NCHUNK, body, 0, unroll=False)
    plsc.subcore_barrier()
    pltpu.sync_copy(acc.at[pl.ds(sid * RPT, RPT)],
                    out_hbm.at[cid, pl.ds(sid * RPT, RPT)])


def _make_sc_deg():
    return pl.kernel(
        _sc_deg_body,
        mesh=_mesh(),
        out_type=jax.ShapeDtypeStruct((2, NPAD, 128), jnp.float32),
        scratch_types=[
            pltpu.VMEM((NCHUNK, CHUNK), jnp.int32),
            pltpu.VMEM((CHUNK, 128), jnp.float32),
            pltpu.VMEM_SHARED((NPAD, 128), jnp.float32),
        ],
    )


def _sc_scat_body(src_hbm, dst_hbm, tab_hbm, zer_hbm, out_hbm,
                  src_v, dst_v, msg, acc, sem):
    cid = lax.axis_index("c")
    sid = lax.axis_index("s")
    wid = cid * 16 + sid
    pltpu.sync_copy(src_hbm.at[wid], src_v)
    pltpu.sync_copy(dst_hbm.at[wid], dst_v)
    pltpu.sync_copy(zer_hbm, acc.at[pl.ds(sid * RPT, RPT)])
    plsc.subcore_barrier()

    def body(c, carry):
        pltpu.async_copy(tab_hbm.at[src_v.at[c]], msg, sem).wait()
        pltpu.sync_copy(msg, acc.at[dst_v.at[c]], add=True)
        return carry

    lax.fori_loop(0, NCHUNK, body, 0, unroll=False)
    plsc.subcore_barrier()
    pltpu.sync_copy(acc.at[pl.ds(sid * RPT, RPT)],
                    out_hbm.at[cid, pl.ds(sid * RPT, RPT)])


def _make_sc_scat():
    return pl.kernel(
        _sc_scat_body,
        mesh=_mesh(),
        out_type=jax.ShapeDtypeStruct((2, NPAD, 128), jnp.float32),
        scratch_types=[
            pltpu.VMEM((NCHUNK, CHUNK), jnp.int32),
            pltpu.VMEM((NCHUNK, CHUNK), jnp.int32),
            pltpu.VMEM((CHUNK, 128), jnp.float32),
            pltpu.VMEM_SHARED((NPAD, 128), jnp.float32),
            pltpu.SemaphoreType.DMA,
        ],
    )


def _tc1_body(x_ref, w1_ref, degp_ref, hs1_ref, dinv_ref):
    hw1 = _dot(x_ref[...], w1_ref[...])
    deg = 1.0 + degp_ref[0, :N, 0:16] + degp_ref[1, :N, 0:16]
    dinv = lax.rsqrt(deg)
    dinv_ref[...] = dinv
    hs1_ref[...] = jnp.concatenate(
        [dinv * hw1, jnp.zeros((N, 112), jnp.float32)], axis=1)


def _tc2_body(accp_ref, hs1_ref, dinv_ref, b1_ref, w2_ref, hs2_ref):
    p = accp_ref[0, :N, 0:16] + accp_ref[1, :N, 0:16]
    dinv = dinv_ref[...]
    h1 = jax.nn.relu(dinv * p + dinv * hs1_ref[:, 0:16] + b1_ref[...])
    hw2 = _dot(h1, w2_ref[...])
    hs2_ref[...] = jnp.concatenate(
        [dinv[:, 0:1] * hw2, jnp.zeros((N, 96), jnp.float32)], axis=1)


def _gru_layer(gi_ref, wbd_hh, bhh2, f_ref, b_ref):
    def step(t, hcat):
        gf = gi_ref[pl.ds(t, 1), 0:192]
        gb = gi_ref[pl.ds(N - 1 - t, 1), 192:384]
        gh = _dot(hcat, wbd_hh)
        G = jnp.concatenate([gf, gb], axis=0)
        GH = jnp.concatenate([gh[:, 0:192], gh[:, 192:384]], axis=0) + bhh2
        r = jax.nn.sigmoid(G[:, 0:64] + GH[:, 0:64])
        z = jax.nn.sigmoid(G[:, 64:128] + GH[:, 64:128])
        nn_ = jnp.tanh(G[:, 128:192] + r * GH[:, 128:192])
        h = jnp.concatenate([hcat[:, 0:64], hcat[:, 64:128]], axis=0)
        hnew = (1.0 - z) * nn_ + z * h
        f_ref[pl.ds(t, 1), :] = hnew[0:1, :]
        b_ref[pl.ds(N - 1 - t, 1), :] = hnew[1:2, :]
        return jnp.concatenate([hnew[0:1, :], hnew[1:2, :]], axis=1)

    lax.fori_loop(0, N, step, jnp.zeros((1, 128), jnp.float32))


def _blockdiag(a, b):
    ka = a.shape[0]
    kb = b.shape[0]
    za = jnp.zeros((ka, 192), jnp.float32)
    zb = jnp.zeros((kb, 192), jnp.float32)
    return jnp.concatenate(
        [jnp.concatenate([a, za], axis=1),
         jnp.concatenate([zb, b], axis=1)], axis=0)


def _gates_body(u_ref, wf_ref, wb_ref, bf_ref, bb_ref, gi_ref):
    wih = jnp.concatenate([wf_ref[...].T, wb_ref[...].T], axis=1)
    bih = jnp.concatenate([bf_ref[...], bb_ref[...]], axis=1)
    gi_ref[...] = _dot(u_ref[...], wih) + bih


def _h2_body(accp_ref, hs2_ref, dinv_ref, b2_ref, h2_ref):
    p = accp_ref[0, :N, 0:32] + accp_ref[1, :N, 0:32]
    dinv = dinv_ref[:, 0:1]
    h2_ref[...] = jax.nn.relu(
        dinv * p + dinv * hs2_ref[:, 0:32] + b2_ref[...])


def _scan_body(gi_ref, wf_hh_ref, wb_hh_ref, bf_hh_ref, bb_hh_ref,
               f_ref, b_ref):
    wbd_hh = _blockdiag(wf_hh_ref[...].T, wb_hh_ref[...].T)
    bhh = jnp.concatenate([bf_hh_ref[...], bb_hh_ref[...]], axis=0)
    _gru_layer(gi_ref, wbd_hh, bhh, f_ref, b_ref)


def _cat_gates_body(f_ref, b_ref, wf_ref, wb_ref, bf_ref, bb_ref, gi_ref):
    u = jnp.concatenate([f_ref[...], b_ref[...]], axis=1)
    wih = jnp.concatenate([wf_ref[...].T, wb_ref[...].T], axis=1)
    bih = jnp.concatenate([bf_ref[...], bb_ref[...]], axis=1)
    gi_ref[...] = _dot(u, wih) + bih


def _head_body(f_ref, b_ref, wl_ref, bl_ref, out_ref):
    u = jnp.concatenate([f_ref[...], b_ref[...]], axis=1)
    out_ref[...] = _dot(u, wl_ref[...]) + bl_ref[...]


@jax.jit
def kernel(x, edge_index, W1, b1, W2, b2,
           l0f_Wih, l0f_Whh, l0f_bih, l0f_bhh,
           l0b_Wih, l0b_Whh, l0b_bih, l0b_bhh,
           l1f_Wih, l1f_Whh, l1f_bih, l1f_bhh,
           l1b_Wih, l1b_Whh, l1b_bih, l1b_bhh,
           Wl, bl):
    src = edge_index[0]
    dst = edge_index[1]
    pad = E_PAD - E
    src_p = jnp.concatenate([src, jnp.zeros((pad,), jnp.int32)])
    dst_p = jnp.concatenate([dst, jnp.full((pad,), NPAD - 1, jnp.int32)])
    src_p = src_p.reshape(NWORK, NCHUNK, CHUNK)
    dst_p = dst_p.reshape(NWORK, NCHUNK, CHUNK)

    ones_m = jnp.ones((CHUNK, 128), jnp.float32)
    zer_m = jnp.zeros((RPT, 128), jnp.float32)

    degp = _make_sc_deg()(dst_p, ones_m, zer_m)

    hs1, dinv = pl.pallas_call(
        _tc1_body,
        out_shape=[jax.ShapeDtypeStruct((N, 128), jnp.float32),
                   jax.ShapeDtypeStruct((N, 16), jnp.float32)],
    )(x, W1, degp)

    accp1 = _make_sc_scat()(src_p, dst_p, hs1, zer_m)

    hs2 = pl.pallas_call(
        _tc2_body,
        out_shape=jax.ShapeDtypeStruct((N, 128), jnp.float32),
    )(accp1, hs1, dinv, b1.reshape(1, 16), W2)

    accp2 = _make_sc_scat()(src_p, dst_p, hs2, zer_m)

    h2 = pl.pallas_call(
        _h2_body,
        out_shape=jax.ShapeDtypeStruct((N, 32), jnp.float32),
    )(accp2, hs2, dinv, b2.reshape(1, 32))

    gi0 = pl.pallas_call(
        _gates_body,
        out_shape=jax.ShapeDtypeStruct((N, 384), jnp.float32),
    )(h2, l0f_Wih, l0b_Wih, l0f_bih.reshape(1, 192), l0b_bih.reshape(1, 192))

    f0, b0 = pl.pallas_call(
        _scan_body,
        out_shape=[jax.ShapeDtypeStruct((N, 64), jnp.float32),
                   jax.ShapeDtypeStruct((N, 64), jnp.float32)],
    )(gi0, l0f_Whh, l0b_Whh, l0f_bhh.reshape(1, 192), l0b_bhh.reshape(1, 192))

    gi1 = pl.pallas_call(
        _cat_gates_body,
        out_shape=jax.ShapeDtypeStruct((N, 384), jnp.float32),
    )(f0, b0, l1f_Wih, l1b_Wih, l1f_bih.reshape(1, 192),
      l1b_bih.reshape(1, 192))

    f1, b1 = pl.pallas_call(
        _scan_body,
        out_shape=[jax.ShapeDtypeStruct((N, 64), jnp.float32),
                   jax.ShapeDtypeStruct((N, 64), jnp.float32)],
    )(gi1, l1f_Whh, l1b_Whh, l1f_bhh.reshape(1, 192), l1b_bhh.reshape(1, 192))

    out = pl.pallas_call(
        _head_body,
        out_shape=jax.ShapeDtypeStruct((N, 10), jnp.float32),
    )(f1, b1, Wl, bl.reshape(1, 10))
    return out


# speedup vs baseline: 8.1823x; 1.5918x over previous
import functools

import jax
import jax.numpy as jnp
from jax import lax
from jax.experimental import pallas as pl
from jax.experimental.pallas import tpu as pltpu
from jax.experimental.pallas import tpu_sc as plsc

N = 10000
E = 320000
NWORK = 32
CHUNK = 128
EPW = (E + NWORK * CHUNK - 1) // (NWORK * CHUNK) * CHUNK
E_PAD = EPW * NWORK
NCHUNK = EPW // CHUNK
NPAD = 10240
RPT = NPAD // 16

def _mesh():
    return plsc.VectorSubcoreMesh(core_axis_name="c", subcore_axis_name="s")


_HIGH = jax.lax.Precision.HIGHEST


def _dot(a, b):
    return jnp.dot(a, b, precision=_HIGH, preferred_element_type=jnp.float32)


def _sc_deg_body(dst_hbm, ones_hbm, zer_hbm, out_hbm, dst_v, ones_v, acc):
    cid = lax.axis_index("c")
    sid = lax.axis_index("s")
    wid = cid * 16 + sid
    pltpu.sync_copy(dst_hbm.at[wid], dst_v)
    pltpu.sync_copy(ones_hbm, ones_v)
    pltpu.sync_copy(zer_hbm, acc.at[pl.ds(sid * RPT, RPT)])
    plsc.subcore_barrier()

    def body(c, carry):
        pltpu.sync_copy(ones_v, acc.at[dst_v.at[c]], add=True)
        return carry

    lax.fori_loop(0, NCHUNK, body, 0, unroll=False)
    plsc.subcore_barrier()
    pltpu.sync_copy(acc.at[pl.ds(sid * RPT, RPT)],
                    out_hbm.at[cid, pl.ds(sid * RPT, RPT)])


def _make_sc_deg():
    return pl.kernel(
        _sc_deg_body,
        mesh=_mesh(),
        out_type=jax.ShapeDtypeStruct((2, NPAD, 128), jnp.float32),
        scratch_types=[
            pltpu.VMEM((NCHUNK, CHUNK), jnp.int32),
            pltpu.VMEM((CHUNK, 128), jnp.float32),
            pltpu.VMEM_SHARED((NPAD, 128), jnp.float32),
        ],
    )


def _sc_scat_body(src_hbm, dst_hbm, tab_hbm, zer_hbm, out_hbm,
                  src_v, dst_v, msg, acc, sem):
    cid = lax.axis_index("c")
    sid = lax.axis_index("s")
    wid = cid * 16 + sid
    pltpu.sync_copy(src_hbm.at[wid], src_v)
    pltpu.sync_copy(dst_hbm.at[wid], dst_v)
    pltpu.sync_copy(zer_hbm, acc.at[pl.ds(sid * RPT, RPT)])
    plsc.subcore_barrier()

    def body(c, carry):
        pltpu.async_copy(tab_hbm.at[src_v.at[c]], msg, sem).wait()
        pltpu.sync_copy(msg, acc.at[dst_v.at[c]], add=True)
        return carry

    lax.fori_loop(0, NCHUNK, body, 0, unroll=False)
    plsc.subcore_barrier()
    pltpu.sync_copy(acc.at[pl.ds(sid * RPT, RPT)],
                    out_hbm.at[cid, pl.ds(sid * RPT, RPT)])


def _make_sc_scat():
    return pl.kernel(
        _sc_scat_body,
        mesh=_mesh(),
        out_type=jax.ShapeDtypeStruct((2, NPAD, 128), jnp.float32),
        scratch_types=[
            pltpu.VMEM((NCHUNK, CHUNK), jnp.int32),
            pltpu.VMEM((NCHUNK, CHUNK), jnp.int32),
            pltpu.VMEM((CHUNK, 128), jnp.float32),
            pltpu.VMEM_SHARED((NPAD, 128), jnp.float32),
            pltpu.SemaphoreType.DMA,
        ],
    )


def _tc1_body(x_ref, w1_ref, degp_ref, hs1_ref, dinv_ref):
    hw1 = _dot(x_ref[...], w1_ref[...])
    deg = 1.0 + degp_ref[0, :N, 0:16] + degp_ref[1, :N, 0:16]
    dinv = lax.rsqrt(deg)
    dinv_ref[...] = dinv
    hs1_ref[...] = jnp.concatenate(
        [dinv * hw1, jnp.zeros((N, 112), jnp.float32)], axis=1)


def _tc2_body(accp_ref, hs1_ref, dinv_ref, b1_ref, w2_ref, hs2_ref):
    p = accp_ref[0, :N, 0:16] + accp_ref[1, :N, 0:16]
    dinv = dinv_ref[...]
    h1 = jax.nn.relu(dinv * p + dinv * hs1_ref[:, 0:16] + b1_ref[...])
    hw2 = _dot(h1, w2_ref[...])
    hs2_ref[...] = jnp.concatenate(
        [dinv[:, 0:1] * hw2, jnp.zeros((N, 96), jnp.float32)], axis=1)


def _gru_layer(gi_ref, wperm, bhh_row, fb_ref):
    def step(t, hcat):
        g = gi_ref[pl.ds(t, 1), :]
        gh = _dot(hcat, wperm) + bhh_row
        s = g + gh
        r = jax.nn.sigmoid(s[:, 0:128])
        z = jax.nn.sigmoid(s[:, 128:256])
        nn_ = jnp.tanh(g[:, 256:384] + r * gh[:, 256:384])
        hnew = (1.0 - z) * nn_ + z * hcat
        fb_ref[pl.ds(t, 1), :] = hnew
        return hnew

    lax.fori_loop(0, N, step, jnp.zeros((1, 128), jnp.float32))


def _interleave6(a, b):
    return jnp.concatenate(
        [a[:, 0:64], b[:, 0:64], a[:, 64:128], b[:, 64:128],
         a[:, 128:192], b[:, 128:192]], axis=1)


def _gates_body(u_ref, uf_ref, wf_ref, wb_ref, bf_ref, bb_ref, gi_ref):
    gf = _dot(u_ref[...], wf_ref[...].T) + bf_ref[...]
    gb = _dot(uf_ref[...], wb_ref[...].T) + bb_ref[...]
    for k in range(3):
        gi_ref[:, 128 * k:128 * k + 64] = gf[:, 64 * k:64 * k + 64]
        gi_ref[:, 128 * k + 64:128 * k + 128] = gb[:, 64 * k:64 * k + 64]


def _h2_body(accp_ref, hs2_ref, dinv_ref, b2_ref, h2_ref):
    p = accp_ref[0, :N, 0:32] + accp_ref[1, :N, 0:32]
    dinv = dinv_ref[:, 0:1]
    h2_ref[...] = jax.nn.relu(
        dinv * p + dinv * hs2_ref[:, 0:32] + b2_ref[...])


def _scan_body(gi_ref, wf_hh_ref, wb_hh_ref, bf_hh_ref, bb_hh_ref, fb_ref):
    wft = wf_hh_ref[...].T
    wbt = wb_hh_ref[...].T
    z64 = jnp.zeros((64, 64), jnp.float32)
    top = jnp.concatenate(
        [wft[:, 0:64], z64, wft[:, 64:128], z64, wft[:, 128:192], z64],
        axis=1)
    bot = jnp.concatenate(
        [z64, wbt[:, 0:64], z64, wbt[:, 64:128], z64, wbt[:, 128:192]],
        axis=1)
    wperm = jnp.concatenate([top, bot], axis=0)
    bhh_row = _interleave6(bf_hh_ref[...], bb_hh_ref[...])
    _gru_layer(gi_ref, wperm, bhh_row, fb_ref)


def _head_body(u_ref, wl_ref, bl_ref, out_ref):
    out_ref[...] = _dot(u_ref[...], wl_ref[...]) + bl_ref[...]


@jax.jit
def kernel(x, edge_index, W1, b1, W2, b2,
           l0f_Wih, l0f_Whh, l0f_bih, l0f_bhh,
           l0b_Wih, l0b_Whh, l0b_bih, l0b_bhh,
           l1f_Wih, l1f_Whh, l1f_bih, l1f_bhh,
           l1b_Wih, l1b_Whh, l1b_bih, l1b_bhh,
           Wl, bl):
    src = edge_index[0]
    dst = edge_index[1]
    pad = E_PAD - E
    src_p = jnp.concatenate([src, jnp.zeros((pad,), jnp.int32)])
    dst_p = jnp.concatenate([dst, jnp.full((pad,), NPAD - 1, jnp.int32)])
    src_p = src_p.reshape(NWORK, NCHUNK, CHUNK)
    dst_p = dst_p.reshape(NWORK, NCHUNK, CHUNK)

    ones_m = jnp.ones((CHUNK, 128), jnp.float32)
    zer_m = jnp.zeros((RPT, 128), jnp.float32)

    degp = _make_sc_deg()(dst_p, ones_m, zer_m)

    hs1, dinv = pl.pallas_call(
        _tc1_body,
        out_shape=[jax.ShapeDtypeStruct((N, 128), jnp.float32),
                   jax.ShapeDtypeStruct((N, 16), jnp.float32)],
    )(x, W1, degp)

    accp1 = _make_sc_scat()(src_p, dst_p, hs1, zer_m)

    hs2 = pl.pallas_call(
        _tc2_body,
        out_shape=jax.ShapeDtypeStruct((N, 128), jnp.float32),
    )(accp1, hs1, dinv, b1.reshape(1, 16), W2)

    accp2 = _make_sc_scat()(src_p, dst_p, hs2, zer_m)

    h2 = pl.pallas_call(
        _h2_body,
        out_shape=jax.ShapeDtypeStruct((N, 32), jnp.float32),
    )(accp2, hs2, dinv, b2.reshape(1, 32))

    def scan_layer(u, wf_ih, wb_ih, bf_ih, bb_ih, wf_hh, wb_hh,
                   bf_hh, bb_hh):
        ku = u.shape[1]
        gi = pl.pallas_call(
            _gates_body,
            grid=(10,),
            in_specs=[
                pl.BlockSpec((N // 10, ku), lambda i: (i, 0)),
                pl.BlockSpec((N // 10, ku), lambda i: (i, 0)),
                pl.BlockSpec((192, ku), lambda i: (0, 0)),
                pl.BlockSpec((192, ku), lambda i: (0, 0)),
                pl.BlockSpec((1, 192), lambda i: (0, 0)),
                pl.BlockSpec((1, 192), lambda i: (0, 0)),
            ],
            out_specs=pl.BlockSpec((N // 10, 384), lambda i: (i, 0)),
            out_shape=jax.ShapeDtypeStruct((N, 384), jnp.float32),
        )(u, jnp.flip(u, axis=0), wf_ih, wb_ih,
          bf_ih.reshape(1, 192), bb_ih.reshape(1, 192))
        fb = pl.pallas_call(
            _scan_body,
            out_shape=jax.ShapeDtypeStruct((N, 128), jnp.float32),
        )(gi, wf_hh, wb_hh, bf_hh.reshape(1, 192), bb_hh.reshape(1, 192))
        return jnp.concatenate(
            [fb[:, 0:64], jnp.flip(fb[:, 64:128], axis=0)], axis=1)

    u0 = scan_layer(h2, l0f_Wih, l0b_Wih, l0f_bih, l0b_bih,
                    l0f_Whh, l0b_Whh, l0f_bhh, l0b_bhh)
    u1 = scan_layer(u0, l1f_Wih, l1b_Wih, l1f_bih, l1b_bih,
                    l1f_Whh, l1b_Whh, l1f_bhh, l1b_bhh)

    out = pl.pallas_call(
        _head_body,
        out_shape=jax.ShapeDtypeStruct((N, 10), jnp.float32),
    )(u1, Wl, bl.reshape(1, 10))
    return out


# speedup vs baseline: 12.9132x; 1.5782x over previous
import functools

import jax
import jax.numpy as jnp
from jax import lax
from jax.experimental import pallas as pl
from jax.experimental.pallas import tpu as pltpu
from jax.experimental.pallas import tpu_sc as plsc

N = 10000
E = 320000
NWORK = 32
CHUNK = 128
EPW = (E + NWORK * CHUNK - 1) // (NWORK * CHUNK) * CHUNK
E_PAD = EPW * NWORK
NCHUNK = EPW // CHUNK
NPAD = 10240
RPT = NPAD // 16

def _mesh():
    return plsc.VectorSubcoreMesh(core_axis_name="c", subcore_axis_name="s")


_HIGH = jax.lax.Precision.HIGHEST


def _dot(a, b):
    return jnp.dot(a, b, precision=_HIGH, preferred_element_type=jnp.float32)


def _sc_deg_body(dst_hbm, ones_hbm, zer_hbm, out_hbm, dst_v, ones_v, acc):
    cid = lax.axis_index("c")
    sid = lax.axis_index("s")
    wid = cid * 16 + sid
    pltpu.sync_copy(dst_hbm.at[wid], dst_v)
    pltpu.sync_copy(ones_hbm, ones_v)
    pltpu.sync_copy(zer_hbm, acc.at[pl.ds(sid * RPT, RPT)])
    plsc.subcore_barrier()

    def body(c, carry):
        pltpu.sync_copy(ones_v, acc.at[dst_v.at[c]], add=True)
        return carry

    lax.fori_loop(0, NCHUNK, body, 0, unroll=False)
    plsc.subcore_barrier()
    pltpu.sync_copy(acc.at[pl.ds(sid * RPT, RPT)],
                    out_hbm.at[cid, pl.ds(sid * RPT, RPT)])


def _make_sc_deg():
    return pl.kernel(
        _sc_deg_body,
        mesh=_mesh(),
        out_type=jax.ShapeDtypeStruct((2, NPAD, 128), jnp.float32),
        scratch_types=[
            pltpu.VMEM((NCHUNK, CHUNK), jnp.int32),
            pltpu.VMEM((CHUNK, 128), jnp.float32),
            pltpu.VMEM_SHARED((NPAD, 128), jnp.float32),
        ],
    )


def _sc_scat_body(src_hbm, dst_hbm, tab_hbm, zer_hbm, out_hbm,
                  src_v, dst_v, msg, acc, sem):
    cid = lax.axis_index("c")
    sid = lax.axis_index("s")
    wid = cid * 16 + sid
    pltpu.sync_copy(src_hbm.at[wid], src_v)
    pltpu.sync_copy(dst_hbm.at[wid], dst_v)
    pltpu.sync_copy(zer_hbm, acc.at[pl.ds(sid * RPT, RPT)])
    plsc.subcore_barrier()

    def body(c, carry):
        pltpu.async_copy(tab_hbm.at[src_v.at[c]], msg, sem).wait()
        pltpu.sync_copy(msg, acc.at[dst_v.at[c]], add=True)
        return carry

    lax.fori_loop(0, NCHUNK, body, 0, unroll=False)
    plsc.subcore_barrier()
    pltpu.sync_copy(acc.at[pl.ds(sid * RPT, RPT)],
                    out_hbm.at[cid, pl.ds(sid * RPT, RPT)])


def _make_sc_scat():
    return pl.kernel(
        _sc_scat_body,
        mesh=_mesh(),
        out_type=jax.ShapeDtypeStruct((2, NPAD, 128), jnp.float32),
        scratch_types=[
            pltpu.VMEM((NCHUNK, CHUNK), jnp.int32),
            pltpu.VMEM((NCHUNK, CHUNK), jnp.int32),
            pltpu.VMEM((CHUNK, 128), jnp.float32),
            pltpu.VMEM_SHARED((NPAD, 128), jnp.float32),
            pltpu.SemaphoreType.DMA,
        ],
    )


def _tc1_body(x_ref, w1_ref, degp_ref, hs1_ref, dinv_ref):
    hw1 = _dot(x_ref[...], w1_ref[...])
    deg = 1.0 + degp_ref[0, :N, 0:16] + degp_ref[1, :N, 0:16]
    dinv = lax.rsqrt(deg)
    dinv_ref[...] = dinv
    hs1_ref[...] = jnp.concatenate(
        [dinv * hw1, jnp.zeros((N, 112), jnp.float32)], axis=1)


def _tc2_body(accp_ref, hs1_ref, dinv_ref, b1_ref, w2_ref, hs2_ref):
    p = accp_ref[0, :N, 0:16] + accp_ref[1, :N, 0:16]
    dinv = dinv_ref[...]
    h1 = jax.nn.relu(dinv * p + dinv * hs1_ref[:, 0:16] + b1_ref[...])
    hw2 = _dot(h1, w2_ref[...])
    hs2_ref[...] = jnp.concatenate(
        [dinv[:, 0:1] * hw2, jnp.zeros((N, 96), jnp.float32)], axis=1)


def _gru_layer(gi_ref, wperm, bhh_row, fb_ref):
    def step(t, hcat):
        g = gi_ref[pl.ds(t, 1), :]
        gh = jnp.dot(hcat, wperm,
                     preferred_element_type=jnp.float32) + bhh_row
        s = g + gh
        r = jax.nn.sigmoid(s[:, 0:128])
        z = jax.nn.sigmoid(s[:, 128:256])
        nn_ = jnp.tanh(g[:, 256:384] + r * gh[:, 256:384])
        hnew = (1.0 - z) * nn_ + z * hcat
        fb_ref[pl.ds(t, 1), :] = hnew
        return hnew

    lax.fori_loop(0, N, step, jnp.zeros((1, 128), jnp.float32),
                  unroll=4)


def _interleave6(a, b):
    return jnp.concatenate(
        [a[:, 0:64], b[:, 0:64], a[:, 64:128], b[:, 64:128],
         a[:, 128:192], b[:, 128:192]], axis=1)


def _gates_body(u_ref, uf_ref, wf_ref, wb_ref, bf_ref, bb_ref, gi_ref):
    gf = _dot(u_ref[...], wf_ref[...].T) + bf_ref[...]
    gb = _dot(uf_ref[...], wb_ref[...].T) + bb_ref[...]
    for k in range(3):
        gi_ref[:, 128 * k:128 * k + 64] = gf[:, 64 * k:64 * k + 64]
        gi_ref[:, 128 * k + 64:128 * k + 128] = gb[:, 64 * k:64 * k + 64]


def _h2_body(accp_ref, hs2_ref, dinv_ref, b2_ref, h2_ref):
    p = accp_ref[0, :N, 0:32] + accp_ref[1, :N, 0:32]
    dinv = dinv_ref[:, 0:1]
    h2_ref[...] = jax.nn.relu(
        dinv * p + dinv * hs2_ref[:, 0:32] + b2_ref[...])


def _scan_body(gi_ref, wf_hh_ref, wb_hh_ref, bf_hh_ref, bb_hh_ref, fb_ref):
    wft = wf_hh_ref[...].T
    wbt = wb_hh_ref[...].T
    z64 = jnp.zeros((64, 64), jnp.float32)
    top = jnp.concatenate(
        [wft[:, 0:64], z64, wft[:, 64:128], z64, wft[:, 128:192], z64],
        axis=1)
    bot = jnp.concatenate(
        [z64, wbt[:, 0:64], z64, wbt[:, 64:128], z64, wbt[:, 128:192]],
        axis=1)
    wperm = jnp.concatenate([top, bot], axis=0)
    bhh_row = _interleave6(bf_hh_ref[...], bb_hh_ref[...])
    _gru_layer(gi_ref, wperm, bhh_row, fb_ref)


def _head_body(u_ref, wl_ref, bl_ref, out_ref):
    out_ref[...] = _dot(u_ref[...], wl_ref[...]) + bl_ref[...]


@jax.jit
def kernel(x, edge_index, W1, b1, W2, b2,
           l0f_Wih, l0f_Whh, l0f_bih, l0f_bhh,
           l0b_Wih, l0b_Whh, l0b_bih, l0b_bhh,
           l1f_Wih, l1f_Whh, l1f_bih, l1f_bhh,
           l1b_Wih, l1b_Whh, l1b_bih, l1b_bhh,
           Wl, bl):
    src = edge_index[0]
    dst = edge_index[1]
    pad = E_PAD - E
    src_p = jnp.concatenate([src, jnp.zeros((pad,), jnp.int32)])
    dst_p = jnp.concatenate([dst, jnp.full((pad,), NPAD - 1, jnp.int32)])
    src_p = src_p.reshape(NWORK, NCHUNK, CHUNK)
    dst_p = dst_p.reshape(NWORK, NCHUNK, CHUNK)

    ones_m = jnp.ones((CHUNK, 128), jnp.float32)
    zer_m = jnp.zeros((RPT, 128), jnp.float32)

    degp = _make_sc_deg()(dst_p, ones_m, zer_m)

    hs1, dinv = pl.pallas_call(
        _tc1_body,
        out_shape=[jax.ShapeDtypeStruct((N, 128), jnp.float32),
                   jax.ShapeDtypeStruct((N, 16), jnp.float32)],
    )(x, W1, degp)

    accp1 = _make_sc_scat()(src_p, dst_p, hs1, zer_m)

    hs2 = pl.pallas_call(
        _tc2_body,
        out_shape=jax.ShapeDtypeStruct((N, 128), jnp.float32),
    )(accp1, hs1, dinv, b1.reshape(1, 16), W2)

    accp2 = _make_sc_scat()(src_p, dst_p, hs2, zer_m)

    h2 = pl.pallas_call(
        _h2_body,
        out_shape=jax.ShapeDtypeStruct((N, 32), jnp.float32),
    )(accp2, hs2, dinv, b2.reshape(1, 32))

    def scan_layer(u, wf_ih, wb_ih, bf_ih, bb_ih, wf_hh, wb_hh,
                   bf_hh, bb_hh):
        ku = u.shape[1]
        gi = pl.pallas_call(
            _gates_body,
            grid=(10,),
            in_specs=[
                pl.BlockSpec((N // 10, ku), lambda i: (i, 0)),
                pl.BlockSpec((N // 10, ku), lambda i: (i, 0)),
                pl.BlockSpec((192, ku), lambda i: (0, 0)),
                pl.BlockSpec((192, ku), lambda i: (0, 0)),
                pl.BlockSpec((1, 192), lambda i: (0, 0)),
                pl.BlockSpec((1, 192), lambda i: (0, 0)),
            ],
            out_specs=pl.BlockSpec((N // 10, 384), lambda i: (i, 0)),
            out_shape=jax.ShapeDtypeStruct((N, 384), jnp.float32),
        )(u, jnp.flip(u, axis=0), wf_ih, wb_ih,
          bf_ih.reshape(1, 192), bb_ih.reshape(1, 192))
        fb = pl.pallas_call(
            _scan_body,
            out_shape=jax.ShapeDtypeStruct((N, 128), jnp.float32),
        )(gi, wf_hh, wb_hh, bf_hh.reshape(1, 192), bb_hh.reshape(1, 192))
        return jnp.concatenate(
            [fb[:, 0:64], jnp.flip(fb[:, 64:128], axis=0)], axis=1)

    u0 = scan_layer(h2, l0f_Wih, l0b_Wih, l0f_bih, l0b_bih,
                    l0f_Whh, l0b_Whh, l0f_bhh, l0b_bhh)
    u1 = scan_layer(u0, l1f_Wih, l1b_Wih, l1f_bih, l1b_bih,
                    l1f_Whh, l1b_Whh, l1f_bhh, l1b_bhh)

    out = pl.pallas_call(
        _head_body,
        out_shape=jax.ShapeDtypeStruct((N, 10), jnp.float32),
    )(u1, Wl, bl.reshape(1, 10))
    return out


# speedup vs baseline: 14.1058x; 1.0924x over previous
import functools

import jax
import jax.numpy as jnp
from jax import lax
from jax.experimental import pallas as pl
from jax.experimental.pallas import tpu as pltpu
from jax.experimental.pallas import tpu_sc as plsc

N = 10000
E = 320000
NWORK = 32
CHUNK = 128
EPW = (E + NWORK * CHUNK - 1) // (NWORK * CHUNK) * CHUNK
E_PAD = EPW * NWORK
NCHUNK = EPW // CHUNK
NPAD = 10240
RPT = NPAD // 16

def _mesh():
    return plsc.VectorSubcoreMesh(core_axis_name="c", subcore_axis_name="s")


_HIGH = jax.lax.Precision.HIGHEST


def _dot(a, b):
    return jnp.dot(a, b, precision=_HIGH, preferred_element_type=jnp.float32)


def _sc_deg_body(dst_hbm, ones_hbm, zer_hbm, out_hbm, dst_v, ones_v, acc):
    cid = lax.axis_index("c")
    sid = lax.axis_index("s")
    wid = cid * 16 + sid
    pltpu.sync_copy(dst_hbm.at[wid], dst_v)
    pltpu.sync_copy(ones_hbm, ones_v)
    pltpu.sync_copy(zer_hbm, acc.at[pl.ds(sid * RPT, RPT)])
    plsc.subcore_barrier()

    def body(c, carry):
        pltpu.sync_copy(ones_v, acc.at[dst_v.at[c]], add=True)
        return carry

    lax.fori_loop(0, NCHUNK, body, 0, unroll=False)
    plsc.subcore_barrier()
    pltpu.sync_copy(acc.at[pl.ds(sid * RPT, RPT)],
                    out_hbm.at[cid, pl.ds(sid * RPT, RPT)])


def _make_sc_deg():
    return pl.kernel(
        _sc_deg_body,
        mesh=_mesh(),
        out_type=jax.ShapeDtypeStruct((2, NPAD, 128), jnp.float32),
        scratch_types=[
            pltpu.VMEM((NCHUNK, CHUNK), jnp.int32),
            pltpu.VMEM((CHUNK, 128), jnp.float32),
            pltpu.VMEM_SHARED((NPAD, 128), jnp.float32),
        ],
    )


PCH = 16


def _sc_scat_body(F, src_hbm, dst_hbm, tab_hbm, zer_hbm, out_hbm,
                  src_v, dst_v, msg0, msg1, acc, sem0, sem1):
    cid = lax.axis_index("c")
    sid = lax.axis_index("s")
    wid = cid * 16 + sid
    pltpu.sync_copy(zer_hbm, acc.at[pl.ds(sid * RPT, RPT)])
    plsc.subcore_barrier()

    def phase(ph, carry):
        pltpu.sync_copy(src_hbm.at[wid, pl.ds(ph * PCH, PCH)], src_v)
        pltpu.sync_copy(dst_hbm.at[wid, pl.ds(ph * PCH, PCH)], dst_v)
        pltpu.async_copy(tab_hbm.at[src_v.at[0]], msg0, sem0)

        def body(c2, carry2):
            for p, msg, sem in ((0, msg0, sem0), (1, msg1, sem1)):
                c = c2 * 2 + p
                msgn, semn = (msg1, sem1) if p == 0 else (msg0, sem0)

                @pl.when(c + 1 < PCH)
                def _():
                    pltpu.async_copy(tab_hbm.at[src_v.at[c + 1]], msgn, semn)

                pltpu.make_async_copy(tab_hbm.at[src_v.at[c]], msg,
                                      sem).wait()
                pltpu.sync_copy(msg, acc.at[dst_v.at[c]], add=True)
            return carry2

        lax.fori_loop(0, PCH // 2, body, 0, unroll=False)
        return carry

    lax.fori_loop(0, NCHUNK // PCH, phase, 0, unroll=False)
    plsc.subcore_barrier()
    pltpu.sync_copy(acc.at[pl.ds(sid * RPT, RPT)],
                    out_hbm.at[cid, pl.ds(sid * RPT, RPT)])


def _make_sc_scat(F):
    return pl.kernel(
        functools.partial(_sc_scat_body, F),
        mesh=_mesh(),
        out_type=jax.ShapeDtypeStruct((2, NPAD, 128), jnp.float32),
        scratch_types=[
            pltpu.VMEM((PCH, CHUNK), jnp.int32),
            pltpu.VMEM((PCH, CHUNK), jnp.int32),
            pltpu.VMEM((CHUNK, 128), jnp.float32),
            pltpu.VMEM((CHUNK, 128), jnp.float32),
            pltpu.VMEM_SHARED((NPAD, 128), jnp.float32),
            pltpu.SemaphoreType.DMA,
            pltpu.SemaphoreType.DMA,
        ],
    )


def _tc1_body(x_ref, w1_ref, degp_ref, hs1_ref, dinv_ref):
    hw1 = _dot(x_ref[...], w1_ref[...])
    deg = 1.0 + degp_ref[0, :N, 0:16] + degp_ref[1, :N, 0:16]
    dinv = lax.rsqrt(deg)
    dinv_ref[...] = dinv
    hs1_ref[...] = jnp.concatenate(
        [dinv * hw1, jnp.zeros((N, 112), jnp.float32)], axis=1)


def _tc2_body(accp_ref, hs1_ref, dinv_ref, b1_ref, w2_ref, hs2_ref):
    p = accp_ref[0, :N, 0:16] + accp_ref[1, :N, 0:16]
    dinv = dinv_ref[...]
    h1 = jax.nn.relu(dinv * p + dinv * hs1_ref[:, 0:16] + b1_ref[...])
    hw2 = _dot(h1, w2_ref[...])
    hs2_ref[...] = jnp.concatenate(
        [dinv[:, 0:1] * hw2, jnp.zeros((N, 96), jnp.float32)], axis=1)


def _gru_layer(gi_ref, wperm, bhh_row, fb_ref):
    def step(t, hcat):
        g = gi_ref[pl.ds(t, 1), :]
        gh = jnp.dot(hcat, wperm,
                     preferred_element_type=jnp.float32) + bhh_row
        s = g + gh
        r = jax.nn.sigmoid(s[:, 0:128])
        z = jax.nn.sigmoid(s[:, 128:256])
        nn_ = jnp.tanh(g[:, 256:384] + r * gh[:, 256:384])
        hnew = (1.0 - z) * nn_ + z * hcat
        fb_ref[pl.ds(t, 1), :] = hnew
        return hnew

    lax.fori_loop(0, N, step, jnp.zeros((1, 128), jnp.float32),
                  unroll=4)


def _interleave6(a, b):
    return jnp.concatenate(
        [a[:, 0:64], b[:, 0:64], a[:, 64:128], b[:, 64:128],
         a[:, 128:192], b[:, 128:192]], axis=1)


def _gates_body(u_ref, uf_ref, wf_ref, wb_ref, bf_ref, bb_ref, gi_ref):
    gf = _dot(u_ref[...], wf_ref[...].T) + bf_ref[...]
    gb = _dot(uf_ref[...], wb_ref[...].T) + bb_ref[...]
    for k in range(3):
        gi_ref[:, 128 * k:128 * k + 64] = gf[:, 64 * k:64 * k + 64]
        gi_ref[:, 128 * k + 64:128 * k + 128] = gb[:, 64 * k:64 * k + 64]


def _h2_body(accp_ref, hs2_ref, dinv_ref, b2_ref, h2_ref):
    p = accp_ref[0, :N, 0:32] + accp_ref[1, :N, 0:32]
    dinv = dinv_ref[:, 0:1]
    h2_ref[...] = jax.nn.relu(
        dinv * p + dinv * hs2_ref[:, 0:32] + b2_ref[...])


def _scan_body(gi_ref, wf_hh_ref, wb_hh_ref, bf_hh_ref, bb_hh_ref, fb_ref):
    wft = wf_hh_ref[...].T
    wbt = wb_hh_ref[...].T
    z64 = jnp.zeros((64, 64), jnp.float32)
    top = jnp.concatenate(
        [wft[:, 0:64], z64, wft[:, 64:128], z64, wft[:, 128:192], z64],
        axis=1)
    bot = jnp.concatenate(
        [z64, wbt[:, 0:64], z64, wbt[:, 64:128], z64, wbt[:, 128:192]],
        axis=1)
    wperm = jnp.concatenate([top, bot], axis=0)
    bhh_row = _interleave6(bf_hh_ref[...], bb_hh_ref[...])
    _gru_layer(gi_ref, wperm, bhh_row, fb_ref)


def _head_body(u_ref, wl_ref, bl_ref, out_ref):
    out_ref[...] = _dot(u_ref[...], wl_ref[...]) + bl_ref[...]


@jax.jit
def kernel(x, edge_index, W1, b1, W2, b2,
           l0f_Wih, l0f_Whh, l0f_bih, l0f_bhh,
           l0b_Wih, l0b_Whh, l0b_bih, l0b_bhh,
           l1f_Wih, l1f_Whh, l1f_bih, l1f_bhh,
           l1b_Wih, l1b_Whh, l1b_bih, l1b_bhh,
           Wl, bl):
    src = edge_index[0]
    dst = edge_index[1]
    pad = E_PAD - E
    src_p = jnp.concatenate([src, jnp.zeros((pad,), jnp.int32)])
    dst_p = jnp.concatenate([dst, jnp.full((pad,), NPAD - 1, jnp.int32)])
    src_p = src_p.reshape(NWORK, NCHUNK, CHUNK)
    dst_p = dst_p.reshape(NWORK, NCHUNK, CHUNK)

    ones_m = jnp.ones((CHUNK, 128), jnp.float32)
    zer_m = jnp.zeros((RPT, 128), jnp.float32)

    degp = _make_sc_deg()(dst_p, ones_m, zer_m)

    hs1, dinv = pl.pallas_call(
        _tc1_body,
        out_shape=[jax.ShapeDtypeStruct((N, 128), jnp.float32),
                   jax.ShapeDtypeStruct((N, 16), jnp.float32)],
    )(x, W1, degp)

    accp1 = _make_sc_scat(16)(src_p, dst_p, hs1, zer_m)

    hs2 = pl.pallas_call(
        _tc2_body,
        out_shape=jax.ShapeDtypeStruct((N, 128), jnp.float32),
    )(accp1, hs1, dinv, b1.reshape(1, 16), W2)

    accp2 = _make_sc_scat(32)(src_p, dst_p, hs2, zer_m)

    h2 = pl.pallas_call(
        _h2_body,
        out_shape=jax.ShapeDtypeStruct((N, 32), jnp.float32),
    )(accp2, hs2, dinv, b2.reshape(1, 32))

    def scan_layer(u, wf_ih, wb_ih, bf_ih, bb_ih, wf_hh, wb_hh,
                   bf_hh, bb_hh):
        ku = u.shape[1]
        gi = pl.pallas_call(
            _gates_body,
            grid=(10,),
            in_specs=[
                pl.BlockSpec((N // 10, ku), lambda i: (i, 0)),
                pl.BlockSpec((N // 10, ku), lambda i: (i, 0)),
                pl.BlockSpec((192, ku), lambda i: (0, 0)),
                pl.BlockSpec((192, ku), lambda i: (0, 0)),
                pl.BlockSpec((1, 192), lambda i: (0, 0)),
                pl.BlockSpec((1, 192), lambda i: (0, 0)),
            ],
            out_specs=pl.BlockSpec((N // 10, 384), lambda i: (i, 0)),
            out_shape=jax.ShapeDtypeStruct((N, 384), jnp.float32),
        )(u, jnp.flip(u, axis=0), wf_ih, wb_ih,
          bf_ih.reshape(1, 192), bb_ih.reshape(1, 192))
        fb = pl.pallas_call(
            _scan_body,
            out_shape=jax.ShapeDtypeStruct((N, 128), jnp.float32),
        )(gi, wf_hh, wb_hh, bf_hh.reshape(1, 192), bb_hh.reshape(1, 192))
        return jnp.concatenate(
            [fb[:, 0:64], jnp.flip(fb[:, 64:128], axis=0)], axis=1)

    u0 = scan_layer(h2, l0f_Wih, l0b_Wih, l0f_bih, l0b_bih,
                    l0f_Whh, l0b_Whh, l0f_bhh, l0b_bhh)
    u1 = scan_layer(u0, l1f_Wih, l1b_Wih, l1f_bih, l1b_bih,
                    l1f_Whh, l1b_Whh, l1f_bhh, l1b_bhh)

    out = pl.pallas_call(
        _head_body,
        out_shape=jax.ShapeDtypeStruct((N, 10), jnp.float32),
    )(u1, Wl, bl.reshape(1, 10))
    return out
